# baseline (speedup 1.0000x reference)
import functools

import jax
import jax.numpy as jnp
from jax import lax
from jax.experimental import pallas as pl
from jax.experimental.pallas import tpu as pltpu
from jax.experimental.pallas import tpu_sc as plsc

B = 4
C = 64
N = 1024
K = 9
R = B * N
E = R * K
EPS = 1e-5


_NCORES = 2
_NSUB = 16
_NW = _NCORES * _NSUB
_RPW = E // _NW
_CHUNK = 128
_NCH = _RPW // _CHUNK


def _sc_gather_body(n_tables, *refs):
    table_refs = refs[:n_tables]
    gidx_hbm = refs[n_tables]
    out_hbm = refs[n_tables + 1]
    idx_v, rows_v, sem = refs[n_tables + 2:]
    wid = lax.axis_index("s") * _NCORES + lax.axis_index("c")
    base = wid * _RPW
    pltpu.sync_copy(gidx_hbm.at[pl.ds(base, _RPW)], idx_v)
    for t in range(n_tables):
        copies = [
            pltpu.async_copy(
                table_refs[t].at[idx_v.at[pl.ds(c * _CHUNK, _CHUNK)]],
                rows_v.at[pl.ds(c * _CHUNK, _CHUNK)],
                sem,
            )
            for c in range(_NCH)
        ]
        for cp in copies:
            cp.wait()
        pltpu.sync_copy(rows_v, out_hbm.at[t, pl.ds(base, _RPW)])


def _sc_gather(tables, gidx):
    n_tables = len(tables)
    mesh = plsc.VectorSubcoreMesh(core_axis_name="c", subcore_axis_name="s")
    return pl.kernel(
        functools.partial(_sc_gather_body, n_tables),
        out_type=jax.ShapeDtypeStruct((n_tables, E, C), jnp.float32),
        mesh=mesh,
        scratch_types=[
            pltpu.VMEM((_RPW,), jnp.int32),
            pltpu.VMEM((_RPW, C), jnp.float32),
            pltpu.SemaphoreType.DMA,
        ],
        compiler_params=pltpu.CompilerParams(use_tc_tiling_on_sc=False),
    )(*tables, gidx)


_RB = 32


def _matT(x, w):
    return lax.dot_general(x, w, (((1,), (1,)), ((), ())),
                           preferred_element_type=jnp.float32)


def _bn_prep(s, q, cnt):
    mu = s / cnt
    var = q / cnt - mu * mu
    inv = lax.rsqrt(var + EPS)
    return mu, inv


def _k1_body(s0_ref, s1_ref, p0w_ref, p0g_ref, p0b_ref,
             p1w_ref, p1g_ref, p1b_ref,
             gidx_ref, s0p_ref, s1p_ref):
    col_iota = lax.broadcasted_iota(jnp.int32, (_RB, N), 1)

    for b in range(B):
        x_full = s0_ref[pl.ds(b * N, N), :]
        sq_full = jnp.sum(x_full * x_full, axis=1)
        for blk in range(N // _RB):
            r0 = blk * _RB
            x_blk = s0_ref[pl.ds(b * N + r0, _RB), :]
            sq_blk = jnp.sum(x_blk * x_blk, axis=1)
            d = (sq_blk[:, None]
                 - 2.0 * lax.dot_general(x_blk, x_full,
                                         (((1,), (1,)), ((), ())),
                                         preferred_element_type=jnp.float32)
                 + sq_full[None, :])
            for j in range(K):
                rmin = jnp.min(d, axis=1)
                amin = jnp.min(
                    jnp.where(d == rmin[:, None], col_iota, N), axis=1)
                d = jnp.where(col_iota == amin[:, None], jnp.float32(1e30), d)
                gidx_ref[j, pl.ds(b * N + r0, _RB)] = amin + b * N

    for src_ref, w_ref, g_ref, b_ref, dst_ref in (
            (s0_ref, p0w_ref, p0g_ref, p0b_ref, s0p_ref),
            (s1_ref, p1w_ref, p1g_ref, p1b_ref, s1p_ref)):
        y = _matT(src_ref[...], w_ref[...])
        mu, inv = _bn_prep(jnp.sum(y, axis=0), jnp.sum(y * y, axis=0),
                           jnp.float32(R))
        scale = inv * g_ref[...]
        dst_ref[...] = jax.nn.relu(
            (y - mu[None, :]) * scale[None, :] + b_ref[...][None, :])


def _k1(s0_rows, s1_rows, p0w, p0g, p0b, p1w, p1g, p1b):
    return pl.pallas_call(
        _k1_body,
        out_shape=(
            jax.ShapeDtypeStruct((K, R), jnp.int32),
            jax.ShapeDtypeStruct((R, C), jnp.float32),
            jax.ShapeDtypeStruct((R, C), jnp.float32),
        ),
    )(s0_rows, s1_rows, p0w, p0g, p0b, p1w, p1g, p1b)


def _opgroup_body(n_states, has_base, emit_t, refs):
    it = iter(refs)
    h_refs = [next(it) for _ in range(n_states)]
    g_refs = [next(it) for _ in range(n_states)]
    cw_ref, cg_ref, cb_ref = next(it), next(it), next(it)
    ew_ref, eg_ref, eb_ref = next(it), next(it), next(it)
    mw_ref, mg_ref, mb_ref = next(it), next(it), next(it)
    weff_ref = next(it)
    base_ref = next(it) if has_base else None
    out_ref = next(it)
    outt_ref = next(it) if emit_t else None

    acc = base_ref[...] if has_base else None

    for oi in range(n_states):
        si = oi
        h = h_refs[si][...]
        w_skip = weff_ref[oi, 1]
        w_conv = weff_ref[oi, 2]
        w_edge = weff_ref[oi, 3]
        w_mr = weff_ref[oi, 4]

        cw = cw_ref[oi]
        ew_r = ew_ref[oi, :, C:]
        ew_d = ew_ref[oi, :, :C] - ew_r
        mw_r = mw_ref[oi, :, C:]
        mw_d = mw_ref[oi, :, :C] - mw_r

        cy = _matT(h, cw)
        cs = jnp.sum(cy, axis=0)
        cq = jnp.sum(cy * cy, axis=0)

        u = _matT(h, ew_d)
        es = jnp.zeros((C,), jnp.float32)
        eq = jnp.zeros((C,), jnp.float32)
        ym = None
        yn = None
        hm = None
        for j in range(K):
            gj = g_refs[si][j]
            hm = gj if hm is None else jnp.maximum(hm, gj)
            yj = _matT(gj, ew_r) + u
            es = es + jnp.sum(yj, axis=0)
            eq = eq + jnp.sum(yj * yj, axis=0)
            ym = yj if ym is None else jnp.maximum(ym, yj)
            yn = yj if yn is None else jnp.minimum(yn, yj)

        my = _matT(h, mw_d) + _matT(hm, mw_r)
        ms = jnp.sum(my, axis=0)
        mq = jnp.sum(my * my, axis=0)

        cmu, cinv = _bn_prep(cs, cq, jnp.float32(R))
        emu, einv = _bn_prep(es, eq, jnp.float32(R * K))
        mmu, minv = _bn_prep(ms, mq, jnp.float32(R))

        cg = cg_ref[oi]
        cb = cb_ref[oi]
        eg = eg_ref[oi]
        eb = eb_ref[oi]
        mg = mg_ref[oi]
        mb = mb_ref[oi]

        c_out = jax.nn.relu(
            (cy - cmu[None, :]) * (cinv * cg)[None, :] + cb[None, :])
        y_ext = jnp.where((eg >= 0)[None, :], ym, yn)
        e_out = jax.nn.relu(
            (y_ext - emu[None, :]) * (einv * eg)[None, :] + eb[None, :])
        m_out = jax.nn.relu(
            (my - mmu[None, :]) * (minv * mg)[None, :] + mb[None, :])
        o = w_skip * h + w_conv * c_out + w_edge * e_out + w_mr * m_out
        acc = o if acc is None else acc + o

    out_ref[...] = acc
    if emit_t:
        for b in range(B):
            outt_ref[b] = jnp.transpose(acc[b * N:(b + 1) * N, :], (1, 0))


def _opgroup(h_list, g_list, cw, cg, cb, ew, eg, eb, mw, mg, mb,
             weff, base=None, emit_t=False):
    n_states = len(h_list)
    has_base = base is not None
    body = functools.partial(_opgroup_body, n_states, has_base, emit_t)

    def wrapped(*refs):
        body(refs)

    n_in = 2 * n_states + 10 + (1 if has_base else 0)
    in_specs = [pl.BlockSpec(memory_space=pltpu.VMEM)] * (n_in - 1)
    in_specs.insert(2 * n_states + 9, pl.BlockSpec(memory_space=pltpu.SMEM))

    args = list(h_list) + list(g_list) + [cw, cg, cb, ew, eg, eb,
                                          mw, mg, mb, weff]
    if has_base:
        args.append(base)

    out_shape = [jax.ShapeDtypeStruct((R, C), jnp.float32)]
    if emit_t:
        out_shape.append(jax.ShapeDtypeStruct((B, C, N), jnp.float32))

    res = pl.pallas_call(
        wrapped,
        out_shape=tuple(out_shape),
        in_specs=in_specs,
    )(*args)
    return res if emit_t else res[0]


def kernel(s0, s1, weights, selected_idxs, pre0_w, pre0_g, pre0_b,
           pre1_w, pre1_g, pre1_b, conv_w, conv_g, conv_b,
           edge_w, edge_g, edge_b, mr_w, mr_g, mr_b):
    s0_rows = jnp.transpose(jnp.squeeze(s0, -1), (0, 2, 1)).reshape(R, C)
    s1_rows = jnp.transpose(jnp.squeeze(s1, -1), (0, 2, 1)).reshape(R, C)

    sel = jnp.asarray(selected_idxs, jnp.int32)
    onehot = (sel[:, None] == jnp.arange(5, dtype=jnp.int32)[None, :])
    weff = jnp.where((sel == -1)[:, None], weights,
                     onehot.astype(jnp.float32))
    weff = jnp.pad(weff, ((0, 0), (0, 3)))

    gidx, s0p, s1p = _k1(s0_rows, s1_rows, pre0_w, pre0_g, pre0_b,
                         pre1_w, pre1_g, pre1_b)
    gidx_flat = gidx.reshape(E)

    g01 = _sc_gather([s0p, s1p], gidx_flat)
    g_s0p = g01[0].reshape(K, R, C)
    g_s1p = g01[1].reshape(K, R, C)

    def params(idxs):
        ii = jnp.asarray(idxs, jnp.int32)
        return (conv_w[ii], conv_g[ii], conv_b[ii],
                edge_w[ii], edge_g[ii], edge_b[ii],
                mr_w[ii], mr_g[ii], mr_b[ii], weff[ii])

    part = _opgroup([s0p], [g_s0p], *params([0]))
    s2, s2_t = _opgroup([s1p], [g_s1p], *params([1]), base=part, emit_t=True)
    g_s2 = _sc_gather([s2], gidx_flat)[0].reshape(K, R, C)

    part = _opgroup([s0p], [g_s0p], *params([2]))
    part = _opgroup([s1p], [g_s1p], *params([3]), base=part)
    s3, s3_t = _opgroup([s2], [g_s2], *params([4]), base=part, emit_t=True)
    g_s3 = _sc_gather([s3], gidx_flat)[0].reshape(K, R, C)

    part = _opgroup([s0p], [g_s0p], *params([5]))
    part = _opgroup([s1p], [g_s1p], *params([6]), base=part)
    part = _opgroup([s2], [g_s2], *params([7]), base=part)
    _, s4_t = _opgroup([s3], [g_s3], *params([8]), base=part, emit_t=True)

    return jnp.concatenate([s2_t, s3_t, s4_t], axis=1)[..., None]
